# baseline (device time: 78148 ns/iter reference)
import jax
import jax.numpy as jnp
from jax import lax
from jax.experimental import pallas as pl
from jax.experimental.pallas import tpu as pltpu

N_DEV = 16


def _silu(y):
    return y * jax.nn.sigmoid(y)


def kernel(x, w_mat):
    m_per, k = x.shape
    _, n_per = w_mat.shape

    def body(x_ref, w_ref, out_ref, comm_ref, send_sems, recv_sems):
        my_pos = lax.axis_index("i")
        left = lax.rem(my_pos - 1 + N_DEV, N_DEV)
        right = lax.rem(my_pos + 1, N_DEV)

        barrier_sem = pltpu.get_barrier_semaphore()
        for nbr in (left, right):
            pl.semaphore_signal(
                barrier_sem, inc=1,
                device_id=(nbr,), device_id_type=pl.DeviceIdType.MESH,
            )
        pl.semaphore_wait(barrier_sem, 2)

        comm_ref[0, :, :] = x_ref[:, :]
        y0 = jnp.dot(x_ref[:, :], w_ref[:, :],
                     preferred_element_type=jnp.float32)
        out_ref[pl.ds(my_pos * m_per, m_per), :] = _silu(y0)

        for h in range(N_DEV - 1):
            rdma = pltpu.make_async_remote_copy(
                src_ref=comm_ref.at[h],
                dst_ref=comm_ref.at[h + 1],
                send_sem=send_sems.at[h],
                recv_sem=recv_sems.at[h],
                device_id=(right,),
                device_id_type=pl.DeviceIdType.MESH,
            )
            rdma.start()
            rdma.wait()

            origin = lax.rem(my_pos - h - 1 + N_DEV, N_DEV)
            y = jnp.dot(comm_ref[h + 1], w_ref[:, :],
                        preferred_element_type=jnp.float32)
            out_ref[pl.ds(origin * m_per, m_per), :] = _silu(y)

    return pl.pallas_call(
        body,
        out_shape=jax.ShapeDtypeStruct((N_DEV * m_per, n_per), jnp.float32),
        in_specs=[
            pl.BlockSpec(memory_space=pltpu.VMEM),
            pl.BlockSpec(memory_space=pltpu.VMEM),
        ],
        out_specs=pl.BlockSpec(memory_space=pltpu.VMEM),
        scratch_shapes=[
            pltpu.VMEM((N_DEV, m_per, k), jnp.float32),
            pltpu.SemaphoreType.DMA((N_DEV - 1,)),
            pltpu.SemaphoreType.DMA((N_DEV - 1,)),
        ],
        compiler_params=pltpu.CompilerParams(collective_id=0),
    )(x, w_mat)


# device time: 47014 ns/iter; 1.6622x vs baseline; 1.6622x over previous
import jax
import jax.numpy as jnp
from jax import lax
from jax.experimental import pallas as pl
from jax.experimental.pallas import tpu as pltpu

N_DEV = 16

CYCLE = (0, 1, 5, 9, 13, 14, 10, 6, 2, 3, 7, 11, 15, 12, 8, 4)
INV = tuple(CYCLE.index(m) for m in range(N_DEV))

N_R = 8
N_L = 7


def _silu(y):
    return y * jax.nn.sigmoid(y)


def kernel(x, w_mat):
    m_per, k = x.shape
    _, n_per = w_mat.shape

    cyc = jnp.array(CYCLE, jnp.int32)
    pos = jnp.array(INV, jnp.int32)[lax.axis_index("i")]
    right = cyc[lax.rem(pos + 1, N_DEV)]
    left = cyc[lax.rem(pos - 1 + N_DEV, N_DEV)]
    nbrs = jnp.stack([left, right])
    orig_r = cyc[lax.rem(pos - 1 - jnp.arange(N_R, dtype=jnp.int32) + N_DEV, N_DEV)]
    orig_l = cyc[lax.rem(pos + 1 + jnp.arange(N_L, dtype=jnp.int32), N_DEV)]

    def body(x_ref, w_ref, nbrs_ref, orr_ref, orl_ref, out_ref,
             comm_r, comm_l, ssr, rsr, ssl, rsl):
        my_pos = lax.axis_index("i")
        lnbr = nbrs_ref[0]
        rnbr = nbrs_ref[1]

        descs_r = []
        for h in range(N_R):
            descs_r.append(pltpu.make_async_remote_copy(
                src_ref=(x_ref if h == 0 else comm_r.at[h - 1]),
                dst_ref=comm_r.at[h],
                send_sem=ssr.at[h],
                recv_sem=rsr.at[h],
                device_id=(rnbr,),
                device_id_type=pl.DeviceIdType.MESH,
            ))
        descs_l = []
        for h in range(N_L):
            descs_l.append(pltpu.make_async_remote_copy(
                src_ref=(x_ref if h == 0 else comm_l.at[h - 1]),
                dst_ref=comm_l.at[h],
                send_sem=ssl.at[h],
                recv_sem=rsl.at[h],
                device_id=(lnbr,),
                device_id_type=pl.DeviceIdType.MESH,
            ))

        barrier_sem = pltpu.get_barrier_semaphore()
        for nbr in (lnbr, rnbr):
            pl.semaphore_signal(
                barrier_sem, inc=1,
                device_id=(nbr,), device_id_type=pl.DeviceIdType.MESH,
            )
        pl.semaphore_wait(barrier_sem, 2)

        descs_r[0].start()
        descs_l[0].start()
        y0 = jnp.dot(x_ref[:, :], w_ref[:, :],
                     preferred_element_type=jnp.float32)
        out_ref[pl.ds(my_pos * m_per, m_per), :] = _silu(y0)

        for h in range(N_R):
            descs_r[h].wait_recv()
            if h + 1 < N_R:
                descs_r[h + 1].start()
            if h < N_L:
                descs_l[h].wait_recv()
                if h + 1 < N_L:
                    descs_l[h + 1].start()
            yr = jnp.dot(comm_r[h], w_ref[:, :],
                         preferred_element_type=jnp.float32)
            out_ref[pl.ds(orr_ref[h] * m_per, m_per), :] = _silu(yr)
            if h < N_L:
                yl = jnp.dot(comm_l[h], w_ref[:, :],
                             preferred_element_type=jnp.float32)
                out_ref[pl.ds(orl_ref[h] * m_per, m_per), :] = _silu(yl)

        for d in descs_r + descs_l:
            d.wait_send()

    return pl.pallas_call(
        body,
        out_shape=jax.ShapeDtypeStruct((N_DEV * m_per, n_per), jnp.float32),
        in_specs=[
            pl.BlockSpec(memory_space=pltpu.VMEM),
            pl.BlockSpec(memory_space=pltpu.VMEM),
            pl.BlockSpec(memory_space=pltpu.SMEM),
            pl.BlockSpec(memory_space=pltpu.SMEM),
            pl.BlockSpec(memory_space=pltpu.SMEM),
        ],
        out_specs=pl.BlockSpec(memory_space=pltpu.VMEM),
        scratch_shapes=[
            pltpu.VMEM((N_R, m_per, k), jnp.float32),
            pltpu.VMEM((N_L, m_per, k), jnp.float32),
            pltpu.SemaphoreType.DMA((N_R,)),
            pltpu.SemaphoreType.DMA((N_R,)),
            pltpu.SemaphoreType.DMA((N_L,)),
            pltpu.SemaphoreType.DMA((N_L,)),
        ],
        compiler_params=pltpu.CompilerParams(collective_id=0),
    )(x, w_mat, nbrs, orig_r, orig_l)


# device time: 38319 ns/iter; 2.0394x vs baseline; 1.2269x over previous
import jax
import jax.numpy as jnp
from jax import lax
from jax.experimental import pallas as pl
from jax.experimental.pallas import tpu as pltpu

N_DEV = 16

CYCLE = (0, 1, 5, 9, 13, 14, 10, 6, 2, 3, 7, 11, 15, 12, 8, 4)
INV = tuple(CYCLE.index(m) for m in range(N_DEV))

N_R = 8
N_L = 7


def _silu(y):
    return y * jax.nn.sigmoid(y)


def kernel(x, w_mat):
    m_per, k = x.shape
    _, n_per = w_mat.shape

    cyc = jnp.array(CYCLE, jnp.int32)
    pos = jnp.array(INV, jnp.int32)[lax.axis_index("i")]
    right = cyc[lax.rem(pos + 1, N_DEV)]
    left = cyc[lax.rem(pos - 1 + N_DEV, N_DEV)]
    nbrs = jnp.stack([left, right])
    orig_r = cyc[lax.rem(pos - 1 - jnp.arange(N_R, dtype=jnp.int32) + N_DEV, N_DEV)]
    orig_l = cyc[lax.rem(pos + 1 + jnp.arange(N_L, dtype=jnp.int32), N_DEV)]

    half = m_per // 2

    def body(x_ref, w_ref, nbrs_ref, orr_ref, orl_ref, out_ref,
             comm_r, comm_l, ssr, rsr, ssl, rsl):
        my_pos = lax.axis_index("i")
        lnbr = nbrs_ref[0]
        rnbr = nbrs_ref[1]

        def mk(src, dst_comm, h, s, send_sems, recv_sems, nbr):
            rows = pl.ds(s * half, half)
            return pltpu.make_async_remote_copy(
                src_ref=src.at[rows, :],
                dst_ref=dst_comm.at[h].at[rows, :],
                send_sem=send_sems.at[h, s],
                recv_sem=recv_sems.at[h, s],
                device_id=(nbr,),
                device_id_type=pl.DeviceIdType.MESH,
            )

        descs_r = [
            [mk(x_ref if h == 0 else comm_r.at[h - 1], comm_r, h, s,
                ssr, rsr, rnbr) for s in range(2)]
            for h in range(N_R)
        ]
        descs_l = [
            [mk(x_ref if h == 0 else comm_l.at[h - 1], comm_l, h, s,
                ssl, rsl, lnbr) for s in range(2)]
            for h in range(N_L)
        ]

        barrier_sem = pltpu.get_barrier_semaphore()
        for nbr in (lnbr, rnbr):
            pl.semaphore_signal(
                barrier_sem, inc=1,
                device_id=(nbr,), device_id_type=pl.DeviceIdType.MESH,
            )
        pl.semaphore_wait(barrier_sem, 2)

        for s in range(2):
            descs_r[0][s].start()
            descs_l[0][s].start()
        y0 = jnp.dot(x_ref[:, :], w_ref[:, :],
                     preferred_element_type=jnp.float32)
        out_ref[pl.ds(my_pos * m_per, m_per), :] = _silu(y0)

        for h in range(N_R):
            for s in range(2):
                descs_r[h][s].wait_recv()
                if h + 1 < N_R:
                    descs_r[h + 1][s].start()
            if h < N_L:
                for s in range(2):
                    descs_l[h][s].wait_recv()
                    if h + 1 < N_L:
                        descs_l[h + 1][s].start()
            yr = jnp.dot(comm_r[h], w_ref[:, :],
                         preferred_element_type=jnp.float32)
            out_ref[pl.ds(orr_ref[h] * m_per, m_per), :] = _silu(yr)
            if h < N_L:
                yl = jnp.dot(comm_l[h], w_ref[:, :],
                             preferred_element_type=jnp.float32)
                out_ref[pl.ds(orl_ref[h] * m_per, m_per), :] = _silu(yl)

        for pair in descs_r + descs_l:
            for d in pair:
                d.wait_send()

    return pl.pallas_call(
        body,
        out_shape=jax.ShapeDtypeStruct((N_DEV * m_per, n_per), jnp.float32),
        in_specs=[
            pl.BlockSpec(memory_space=pltpu.VMEM),
            pl.BlockSpec(memory_space=pltpu.VMEM),
            pl.BlockSpec(memory_space=pltpu.SMEM),
            pl.BlockSpec(memory_space=pltpu.SMEM),
            pl.BlockSpec(memory_space=pltpu.SMEM),
        ],
        out_specs=pl.BlockSpec(memory_space=pltpu.VMEM),
        scratch_shapes=[
            pltpu.VMEM((N_R, m_per, k), jnp.float32),
            pltpu.VMEM((N_L, m_per, k), jnp.float32),
            pltpu.SemaphoreType.DMA((N_R, 2)),
            pltpu.SemaphoreType.DMA((N_R, 2)),
            pltpu.SemaphoreType.DMA((N_L, 2)),
            pltpu.SemaphoreType.DMA((N_L, 2)),
        ],
        compiler_params=pltpu.CompilerParams(collective_id=0),
    )(x, w_mat, nbrs, orig_r, orig_l)
